# baseline (speedup 1.0000x reference)
import jax
import jax.numpy as jnp
from jax import lax
from jax.experimental import pallas as pl
from jax.experimental.pallas import tpu as pltpu
from jax.experimental.pallas import tpu_sc as plsc

N_FIELDS = 26
VOCAB = 100000
EMBED_DIM = 32
BATCH = 16384

_INFO = plsc.get_sparse_core_info()
_NC, _NS, _NL = _INFO.num_cores, _INFO.num_subcores, _INFO.num_lanes

_ICH = 4096
_OCH = 4096
_VCH = 25088
_VALIGNED = 99968
_CHUNKS = ((0, _VCH), (_VCH, _VCH), (2 * _VCH, _VCH),
           (3 * _VCH, _VALIGNED - 3 * _VCH))

_mesh = plsc.VectorSubcoreMesh(core_axis_name="c", subcore_axis_name="s")


def _body(*refs):
    feats = refs[:N_FIELDS]
    wts = refs[N_FIELDS:2 * N_FIELDS]
    tails = refs[2 * N_FIELDS:3 * N_FIELDS]
    out = refs[3 * N_FIELDS]
    spm_tab, spm_out, tail_v, idx_v, orow_v = refs[3 * N_FIELDS + 1:]

    def _scoped(row_v):
        _run(feats, wts, tails, out, spm_tab, spm_out, row_v, tail_v, idx_v,
             orow_v)

    pl.run_scoped(_scoped, pltpu.VMEM((VOCAB,), jnp.float32))


def _run(feats, wts, tails, out, spm_tab, spm_out, row_v, tail_v, idx_v,
         orow_v):
    c = lax.axis_index("c")
    s = lax.axis_index("s")

    for i in range(N_FIELDS):
        for h in range(2):
            for (off, n) in _CHUNKS:
                @pl.when(s == 0)
                def _stage():
                    pltpu.sync_copy(
                        wts[i].at[pl.ds(_NS * c + 8 * h, 8), pl.ds(off, n)],
                        spm_tab.at[slice(None), pl.ds(0, n)],
                    )

                plsc.subcore_barrier()

                @pl.when((s >= 8 * h) & (s < 8 * h + 8))
                def _pull():
                    pltpu.sync_copy(
                        spm_tab.at[s - 8 * h, pl.ds(0, n)],
                        row_v.at[pl.ds(off, n)],
                    )

                plsc.subcore_barrier()

        pltpu.sync_copy(tails[i], tail_v)
        f = _NS * c + s
        for q in range(2):
            row_v[pl.ds(_VALIGNED + q * _NL, _NL)] = tail_v[f, pl.ds(q * _NL, _NL)]

        for half in range(BATCH // _OCH):
            for ch in range(_OCH // _ICH):
                base = half * _OCH + ch * _ICH
                pltpu.sync_copy(feats[i].at[pl.ds(base, _ICH)], idx_v)

                def _gather(k, _):
                    vec = idx_v[pl.ds(k * _NL, _NL)]
                    orow_v[pl.ds(ch * _ICH + k * _NL, _NL)] = plsc.load_gather(
                        row_v, [vec]
                    )
                    return 0

                lax.fori_loop(0, _ICH // _NL, _gather, 0, unroll=8)

            pltpu.sync_copy(orow_v, spm_out.at[s])
            plsc.subcore_barrier()

            @pl.when(s == 0)
            def _flush():
                pltpu.sync_copy(
                    spm_out,
                    out.at[pl.ds(EMBED_DIM * i + _NS * c, _NS),
                           pl.ds(half * _OCH, _OCH)],
                )

            plsc.subcore_barrier()


_sc_call = pl.kernel(
    _body,
    out_type=jax.ShapeDtypeStruct((N_FIELDS * EMBED_DIM, BATCH), jnp.float32),
    mesh=_mesh,
    scratch_types=[
        pltpu.VMEM_SHARED((8, _VCH), jnp.float32),
        pltpu.VMEM_SHARED((_NS, _OCH), jnp.float32),
        pltpu.VMEM((EMBED_DIM, 2 * _NL), jnp.float32),
        pltpu.VMEM((_ICH,), jnp.int32),
        pltpu.VMEM((_OCH,), jnp.float32),
    ],
    compiler_params=pltpu.CompilerParams(
        use_tc_tiling_on_sc=True,
        needs_layout_passes=False,
        internal_scratch_in_bytes=0,
    ),
)


def kernel(feat_0, feat_1, feat_2, feat_3, feat_4, feat_5, feat_6, feat_7, feat_8, feat_9, feat_10, feat_11, feat_12, feat_13, feat_14, feat_15, feat_16, feat_17, feat_18, feat_19, feat_20, feat_21, feat_22, feat_23, feat_24, feat_25, W_0, W_1, W_2, W_3, W_4, W_5, W_6, W_7, W_8, W_9, W_10, W_11, W_12, W_13, W_14, W_15, W_16, W_17, W_18, W_19, W_20, W_21, W_22, W_23, W_24, W_25):
    feats = (feat_0, feat_1, feat_2, feat_3, feat_4, feat_5, feat_6, feat_7,
             feat_8, feat_9, feat_10, feat_11, feat_12, feat_13, feat_14,
             feat_15, feat_16, feat_17, feat_18, feat_19, feat_20, feat_21,
             feat_22, feat_23, feat_24, feat_25)
    tabs = (W_0, W_1, W_2, W_3, W_4, W_5, W_6, W_7, W_8, W_9, W_10, W_11,
            W_12, W_13, W_14, W_15, W_16, W_17, W_18, W_19, W_20, W_21,
            W_22, W_23, W_24, W_25)
    wts = tuple(w.T for w in tabs)
    tails = tuple(w[_VALIGNED:, :].T for w in tabs)
    out_t = _sc_call(*feats, *wts, *tails)
    return out_t.T
